# baseline (speedup 1.0000x reference)
import jax
import jax.numpy as jnp
from jax.experimental import pallas as pl
from jax.experimental.pallas import tpu as pltpu

_MASK_VAL = -9e15
_LOG2E = 1.4426950408889634
_HEADS = 4
_CHUNK = 256


def _project(x, w, a_src, a_dst, wh_scr, es_scr, ed_scr, h, n, f_out):
    wh = jnp.dot(x, w, preferred_element_type=jnp.float32)
    wh_scr[h, :, :f_out] = wh.astype(jnp.bfloat16)
    col = jax.lax.broadcasted_iota(jnp.int32, (n, f_out), 1)
    wh_scr[h, :, f_out:] = jnp.where(col == 0, 1.0, 0.0).astype(jnp.bfloat16)
    es = jax.lax.dot_general(wh, a_src, (((1,), (1,)), ((), ())),
                             preferred_element_type=jnp.float32)
    es_scr[h] = es.astype(jnp.bfloat16)
    ed = jax.lax.dot_general(a_dst, wh, (((1,), (1,)), ((), ())),
                             preferred_element_type=jnp.float32)
    ed_scr[h] = ed.astype(jnp.bfloat16)


def _head_chunk(wh_scr, es_scr, ed_scr, madd_c, h, r0, f_out):
    e = es_scr[h, pl.ds(r0, _CHUNK), :] + ed_scr[h]
    e = jnp.maximum(e, jnp.bfloat16(0.2) * e) + madd_c
    m = jnp.max(e, axis=1, keepdims=True)
    p = jnp.exp2(e - m)
    hp_aug = jnp.dot(p, wh_scr[h],
                     preferred_element_type=jnp.float32)
    return hp_aug[:, :f_out] / hp_aug[:, f_out:f_out + 1]


def _gat_kernel(x_ref, adj_ref, w1_ref, a1_ref, w2_ref, a2_ref, out_ref,
                wh_scr, es_scr, ed_scr, madd_scr, x_scr):
    layer = pl.program_id(1)
    n = x_ref.shape[1]
    f_out = w2_ref.shape[-1]

    @pl.when(layer == 0)
    def _hidden_layer():
        madd_scr[...] = ((adj_ref[0].astype(jnp.bfloat16) - jnp.bfloat16(1.0))
                         * jnp.bfloat16(-_MASK_VAL))
        x = x_ref[0]
        for h in range(_HEADS):
            _project(x, w1_ref[h], a1_ref[h, :, :f_out], a1_ref[h, :, f_out:],
                     wh_scr, es_scr, ed_scr, h, n, f_out)
        for c in range(n // _CHUNK):
            r0 = c * _CHUNK
            madd_c = madd_scr[pl.ds(r0, _CHUNK), :]
            acc = jnp.zeros((_CHUNK, f_out), jnp.float32)
            for h in range(_HEADS):
                hp = _head_chunk(wh_scr, es_scr, ed_scr, madd_c, h, r0, f_out)
                hp = jnp.where(hp > 0, hp, jnp.exp(hp) - 1.0)
                acc = acc + hp
            x_scr[pl.ds(r0, _CHUNK), :] = acc * (1.0 / _HEADS)

    @pl.when(layer == 1)
    def _output_layer():
        _project(x_scr[...], w2_ref[0], a2_ref[0, :, :f_out],
                 a2_ref[0, :, f_out:], wh_scr, es_scr, ed_scr, 0, n, f_out)
        for c in range(n // _CHUNK):
            r0 = c * _CHUNK
            madd_c = madd_scr[pl.ds(r0, _CHUNK), :]
            hp = _head_chunk(wh_scr, es_scr, ed_scr, madd_c, 0, r0, f_out)
            out_ref[0, pl.ds(r0, _CHUNK), :] = jnp.maximum(hp, 0.0)


def kernel(input_feature, adj, W1, a1, W_out, a_out):
    b, n, f_in = input_feature.shape
    heads, _, f_out = W1.shape
    a1_t = jnp.transpose(a1, (0, 2, 1)) * _LOG2E
    w2 = W_out[None]
    a2_t = jnp.transpose(a_out, (1, 0))[None] * _LOG2E
    return pl.pallas_call(
        _gat_kernel,
        grid=(b, 2),
        in_specs=[
            pl.BlockSpec((1, n, f_in), lambda i, l: (i, 0, 0)),
            pl.BlockSpec((1, n, n), lambda i, l: (i, 0, 0)),
            pl.BlockSpec((heads, f_in, f_out), lambda i, l: (0, 0, 0)),
            pl.BlockSpec((heads, 1, 2 * f_out), lambda i, l: (0, 0, 0)),
            pl.BlockSpec((1, f_out, f_out), lambda i, l: (0, 0, 0)),
            pl.BlockSpec((1, 1, 2 * f_out), lambda i, l: (0, 0, 0)),
        ],
        out_specs=pl.BlockSpec((1, n, f_out), lambda i, l: (i, 0, 0)),
        out_shape=jax.ShapeDtypeStruct((b, n, f_out), jnp.float32),
        scratch_shapes=[
            pltpu.VMEM((heads, n, 2 * f_out), jnp.bfloat16),
            pltpu.VMEM((heads, n, 1), jnp.bfloat16),
            pltpu.VMEM((heads, 1, n), jnp.bfloat16),
            pltpu.VMEM((n, n), jnp.bfloat16),
            pltpu.VMEM((n, f_out), jnp.float32),
        ],
        compiler_params=pltpu.CompilerParams(
            dimension_semantics=("parallel", "arbitrary")),
    )(input_feature, adj, W1, a1_t, w2, a2_t)


# speedup vs baseline: 1.1146x; 1.1146x over previous
import jax
import jax.numpy as jnp
from jax.experimental import pallas as pl
from jax.experimental.pallas import tpu as pltpu

_MASK_VAL = -9e15
_LOG2E = 1.4426950408889634
_HEADS = 4


def _attention_pass(wh_scr, es_scr, ed_scr, madd_scr, h, n, f_out):
    e = es_scr[h] + ed_scr[h]
    e = jnp.maximum(e, jnp.bfloat16(0.2) * e) + madd_scr[...]
    m = jnp.max(e, axis=1, keepdims=True)
    p = jnp.exp2(e - m)
    return jnp.dot(p, wh_scr[h], preferred_element_type=jnp.float32)


def _project(x, w, a_src, a_dst, wh_scr, es_scr, ed_scr, h, n, f_out):
    wh = jnp.dot(x, w, preferred_element_type=jnp.float32)
    wh_scr[h, :, :f_out] = wh.astype(jnp.bfloat16)
    es = jax.lax.dot_general(wh, a_src, (((1,), (1,)), ((), ())),
                             preferred_element_type=jnp.float32)
    es_scr[h] = es.astype(jnp.bfloat16)
    ed = jax.lax.dot_general(a_dst, wh, (((1,), (1,)), ((), ())),
                             preferred_element_type=jnp.float32)
    ed_scr[h] = ed.astype(jnp.bfloat16)


def _gat_kernel(x_ref, adj_ref, w1_ref, a1_ref, w2_ref, a2_ref, out_ref,
                wh_scr, es_scr, ed_scr, madd_scr, x_scr):
    layer = pl.program_id(1)
    n = x_ref.shape[1]
    f_out = w2_ref.shape[-1]

    @pl.when((pl.program_id(0) == 0) & (layer == 0))
    def _init_ones_cols():
        col = jax.lax.broadcasted_iota(jnp.int32, (n, f_out), 1)
        ones_first_col = jnp.where(col == 0, 1.0, 0.0).astype(jnp.bfloat16)
        for h in range(_HEADS):
            wh_scr[h, :, f_out:] = ones_first_col

    @pl.when(layer == 0)
    def _hidden_layer():
        madd_scr[...] = ((adj_ref[0].astype(jnp.bfloat16) - jnp.bfloat16(1.0))
                         * jnp.bfloat16(-_MASK_VAL))
        x = x_ref[0]
        for h in range(_HEADS):
            _project(x, w1_ref[h], a1_ref[h, :, :f_out], a1_ref[h, :, f_out:],
                     wh_scr, es_scr, ed_scr, h, n, f_out)
        acc = jnp.zeros((n, f_out), jnp.float32)
        for h in range(_HEADS):
            hp_aug = _attention_pass(wh_scr, es_scr, ed_scr, madd_scr,
                                     h, n, f_out)
            hp = hp_aug[:, :f_out] / hp_aug[:, f_out:f_out + 1]
            hp = jnp.where(hp > 0, hp, jnp.exp(hp) - 1.0)
            acc = acc + hp
        x_scr[...] = acc * (1.0 / _HEADS)

    @pl.when(layer == 1)
    def _output_layer():
        _project(x_scr[...], w2_ref[0], a2_ref[0, :, :f_out],
                 a2_ref[0, :, f_out:], wh_scr, es_scr, ed_scr, 0, n, f_out)
        hp_aug = _attention_pass(wh_scr, es_scr, ed_scr, madd_scr,
                                 0, n, f_out)
        hp = hp_aug[:, :f_out] / hp_aug[:, f_out:f_out + 1]
        out_ref[0] = jnp.maximum(hp, 0.0)


def kernel(input_feature, adj, W1, a1, W_out, a_out):
    b, n, f_in = input_feature.shape
    heads, _, f_out = W1.shape
    a1_t = jnp.transpose(a1, (0, 2, 1)) * _LOG2E
    w2 = W_out[None]
    a2_t = jnp.transpose(a_out, (1, 0))[None] * _LOG2E
    return pl.pallas_call(
        _gat_kernel,
        grid=(b, 2),
        in_specs=[
            pl.BlockSpec((1, n, f_in), lambda i, l: (i, 0, 0)),
            pl.BlockSpec((1, n, n), lambda i, l: (i, 0, 0)),
            pl.BlockSpec((heads, f_in, f_out), lambda i, l: (0, 0, 0)),
            pl.BlockSpec((heads, 1, 2 * f_out), lambda i, l: (0, 0, 0)),
            pl.BlockSpec((1, f_out, f_out), lambda i, l: (0, 0, 0)),
            pl.BlockSpec((1, 1, 2 * f_out), lambda i, l: (0, 0, 0)),
        ],
        out_specs=pl.BlockSpec((1, n, f_out), lambda i, l: (i, 0, 0)),
        out_shape=jax.ShapeDtypeStruct((b, n, f_out), jnp.float32),
        scratch_shapes=[
            pltpu.VMEM((heads, n, 2 * f_out), jnp.bfloat16),
            pltpu.VMEM((heads, n, 1), jnp.bfloat16),
            pltpu.VMEM((heads, 1, n), jnp.bfloat16),
            pltpu.VMEM((n, n), jnp.bfloat16),
            pltpu.VMEM((n, f_out), jnp.float32),
        ],
        compiler_params=pltpu.CompilerParams(
            dimension_semantics=("arbitrary", "arbitrary")),
    )(input_feature, adj, W1, a1_t, w2, a2_t)


# speedup vs baseline: 1.1198x; 1.0046x over previous
import jax
import jax.numpy as jnp
from jax.experimental import pallas as pl
from jax.experimental.pallas import tpu as pltpu

_MASK_VAL = -9e15
_LOG2E = 1.4426950408889634
_HEADS = 4


def _attention_pass(wh_scr, es_scr, ed_scr, madd_scr, h, n, f_out):
    e = es_scr[h] + ed_scr[h]
    e = jnp.maximum(e, jnp.bfloat16(0.2) * e) + madd_scr[...]
    m = jnp.max(e, axis=1, keepdims=True)
    p = jnp.exp2(e - m)
    return jnp.dot(p, wh_scr[h], preferred_element_type=jnp.float32)


def _project(x, w, a_src, a_dst, wh_scr, es_scr, ed_scr, h, n, f_out):
    wh = jnp.dot(x, w, preferred_element_type=jnp.float32)
    wh_scr[h, :, :f_out] = wh.astype(jnp.bfloat16)
    es = jax.lax.dot_general(wh, a_src, (((1,), (1,)), ((), ())),
                             preferred_element_type=jnp.float32)
    es_scr[h] = es.astype(jnp.bfloat16)
    ed = jax.lax.dot_general(a_dst, wh, (((1,), (1,)), ((), ())),
                             preferred_element_type=jnp.float32)
    ed_scr[h] = ed.astype(jnp.bfloat16)


def _gat_kernel(x_ref, adj_ref, w1_ref, a1_ref, w2_ref, a2_ref, out_ref,
                wh_scr, es_scr, ed_scr, madd_scr, x_scr, e_scr):
    layer = pl.program_id(1)
    n = x_ref.shape[1]
    f_out = w2_ref.shape[-1]

    @pl.when((pl.program_id(0) == 0) & (layer == 0))
    def _init_ones_cols():
        col = jax.lax.broadcasted_iota(jnp.int32, (n, f_out), 1)
        ones_first_col = jnp.where(col == 0, 1.0, 0.0).astype(jnp.bfloat16)
        for h in range(_HEADS):
            wh_scr[h, :, f_out:] = ones_first_col

    @pl.when(layer == 0)
    def _hidden_layer():
        madd_scr[...] = ((adj_ref[0].astype(jnp.bfloat16) - jnp.bfloat16(1.0))
                         * jnp.bfloat16(-_MASK_VAL))
        x = x_ref[0]
        for h in range(_HEADS):
            _project(x, w1_ref[h], a1_ref[h, :, :f_out], a1_ref[h, :, f_out:],
                     wh_scr, es_scr, ed_scr, h, n, f_out)
        madd = madd_scr[...]
        for h in range(_HEADS):
            e = es_scr[h] + ed_scr[h]
            e_scr[h] = jnp.maximum(e, jnp.bfloat16(0.2) * e) + madd
        for h in range(_HEADS):
            m = jnp.max(e_scr[h], axis=1, keepdims=True)
            e_scr[h] = jnp.exp2(e_scr[h] - m)
        acc = jnp.zeros((n, f_out), jnp.float32)
        for h in range(_HEADS):
            hp_aug = jnp.dot(e_scr[h], wh_scr[h],
                             preferred_element_type=jnp.float32)
            hp = hp_aug[:, :f_out] / hp_aug[:, f_out:f_out + 1]
            hp = jnp.where(hp > 0, hp, jnp.exp(hp) - 1.0)
            acc = acc + hp
        x_scr[...] = acc * (1.0 / _HEADS)

    @pl.when(layer == 1)
    def _output_layer():
        _project(x_scr[...], w2_ref[0], a2_ref[0, :, :f_out],
                 a2_ref[0, :, f_out:], wh_scr, es_scr, ed_scr, 0, n, f_out)
        hp_aug = _attention_pass(wh_scr, es_scr, ed_scr, madd_scr,
                                 0, n, f_out)
        hp = hp_aug[:, :f_out] / hp_aug[:, f_out:f_out + 1]
        out_ref[0] = jnp.maximum(hp, 0.0)


def kernel(input_feature, adj, W1, a1, W_out, a_out):
    b, n, f_in = input_feature.shape
    heads, _, f_out = W1.shape
    a1_t = jnp.transpose(a1, (0, 2, 1)) * _LOG2E
    w2 = W_out[None]
    a2_t = jnp.transpose(a_out, (1, 0))[None] * _LOG2E
    return pl.pallas_call(
        _gat_kernel,
        grid=(b, 2),
        in_specs=[
            pl.BlockSpec((1, n, f_in), lambda i, l: (i, 0, 0)),
            pl.BlockSpec((1, n, n), lambda i, l: (i, 0, 0)),
            pl.BlockSpec((heads, f_in, f_out), lambda i, l: (0, 0, 0)),
            pl.BlockSpec((heads, 1, 2 * f_out), lambda i, l: (0, 0, 0)),
            pl.BlockSpec((1, f_out, f_out), lambda i, l: (0, 0, 0)),
            pl.BlockSpec((1, 1, 2 * f_out), lambda i, l: (0, 0, 0)),
        ],
        out_specs=pl.BlockSpec((1, n, f_out), lambda i, l: (i, 0, 0)),
        out_shape=jax.ShapeDtypeStruct((b, n, f_out), jnp.float32),
        scratch_shapes=[
            pltpu.VMEM((heads, n, 2 * f_out), jnp.bfloat16),
            pltpu.VMEM((heads, n, 1), jnp.bfloat16),
            pltpu.VMEM((heads, 1, n), jnp.bfloat16),
            pltpu.VMEM((n, n), jnp.bfloat16),
            pltpu.VMEM((n, f_out), jnp.float32),
            pltpu.VMEM((_HEADS, n, n), jnp.bfloat16),
        ],
        compiler_params=pltpu.CompilerParams(
            dimension_semantics=("arbitrary", "arbitrary")),
    )(input_feature, adj, W1, a1_t, w2, a2_t)


# speedup vs baseline: 1.1429x; 1.0207x over previous
import jax
import jax.numpy as jnp
from jax.experimental import pallas as pl
from jax.experimental.pallas import tpu as pltpu

_MASK_VAL = -9e15
_LOG2E = 1.4426950408889634
_HEADS = 4


def _attention_pass(wh_scr, es_scr, ed_scr, madd_scr, h, n, f_out):
    e = es_scr[h] + ed_scr[h]
    e = jnp.maximum(e, jnp.bfloat16(0.2) * e) + madd_scr[...]
    m = jnp.max(e, axis=1, keepdims=True)
    p = jnp.exp2(e - m)
    return jnp.dot(p, wh_scr[h], preferred_element_type=jnp.float32)


def _project(x, w, av, wh_scr, es_scr, ed_scr, h, n, f_out):
    wh = jnp.dot(x, w, preferred_element_type=jnp.float32)
    wh_scr[h, :, :f_out] = wh.astype(jnp.bfloat16)
    es = jax.lax.dot_general(x, av[0:1], (((1,), (1,)), ((), ())),
                             preferred_element_type=jnp.float32)
    es_scr[h] = es.astype(jnp.bfloat16)
    ed = jax.lax.dot_general(av[1:2], x, (((1,), (1,)), ((), ())),
                             preferred_element_type=jnp.float32)
    ed_scr[h] = ed.astype(jnp.bfloat16)


def _gat_kernel(x_ref, adj_ref, w1_ref, av1_ref, w2_ref, av2_ref, out_ref,
                wh_scr, es_scr, ed_scr, madd_scr, x_scr, e_scr):
    layer = pl.program_id(1)
    n = x_ref.shape[1]
    f_out = w2_ref.shape[-1]

    @pl.when((pl.program_id(0) == 0) & (layer == 0))
    def _init_ones_cols():
        col = jax.lax.broadcasted_iota(jnp.int32, (n, f_out), 1)
        ones_first_col = jnp.where(col == 0, 1.0, 0.0).astype(jnp.bfloat16)
        for h in range(_HEADS):
            wh_scr[h, :, f_out:] = ones_first_col

    @pl.when(layer == 0)
    def _hidden_layer():
        madd_scr[...] = ((adj_ref[0].astype(jnp.bfloat16) - jnp.bfloat16(1.0))
                         * jnp.bfloat16(-_MASK_VAL))
        x = x_ref[0]
        for h in range(_HEADS):
            _project(x, w1_ref[h], av1_ref[h], wh_scr, es_scr, ed_scr,
                     h, n, f_out)
        madd = madd_scr[...]
        for h in range(_HEADS):
            e = es_scr[h] + ed_scr[h]
            e_scr[h] = jnp.maximum(e, jnp.bfloat16(0.2) * e) + madd
        for h in range(_HEADS):
            m = jnp.max(e_scr[h], axis=1, keepdims=True)
            e_scr[h] = jnp.exp2(e_scr[h] - m)
        acc = jnp.zeros((n, f_out), jnp.float32)
        for h in range(_HEADS):
            hp_aug = jnp.dot(e_scr[h], wh_scr[h],
                             preferred_element_type=jnp.float32)
            hp = hp_aug[:, :f_out] / hp_aug[:, f_out:f_out + 1]
            hp = jnp.where(hp > 0, hp, jnp.exp(hp) - 1.0)
            acc = acc + hp
        x_scr[...] = acc * (1.0 / _HEADS)

    @pl.when(layer == 1)
    def _output_layer():
        _project(x_scr[...], w2_ref[0], av2_ref[0], wh_scr, es_scr, ed_scr,
                 0, n, f_out)
        hp_aug = _attention_pass(wh_scr, es_scr, ed_scr, madd_scr,
                                 0, n, f_out)
        hp = hp_aug[:, :f_out] / hp_aug[:, f_out:f_out + 1]
        out_ref[0] = jnp.maximum(hp, 0.0)


def kernel(input_feature, adj, W1, a1, W_out, a_out):
    b, n, f_in = input_feature.shape
    heads, _, f_out = W1.shape
    av1 = jnp.stack([jnp.einsum('hfo,ho->hf', W1, a1[:, :f_out, 0]),
                     jnp.einsum('hfo,ho->hf', W1, a1[:, f_out:, 0])],
                    axis=1) * _LOG2E
    w2 = W_out[None]
    av2 = jnp.stack([W_out @ a_out[:f_out, 0],
                     W_out @ a_out[f_out:, 0]])[None] * _LOG2E
    return pl.pallas_call(
        _gat_kernel,
        grid=(b, 2),
        in_specs=[
            pl.BlockSpec((1, n, f_in), lambda i, l: (i, 0, 0)),
            pl.BlockSpec((1, n, n), lambda i, l: (i, 0, 0)),
            pl.BlockSpec((heads, f_in, f_out), lambda i, l: (0, 0, 0)),
            pl.BlockSpec((heads, 2, f_in), lambda i, l: (0, 0, 0)),
            pl.BlockSpec((1, f_out, f_out), lambda i, l: (0, 0, 0)),
            pl.BlockSpec((1, 2, f_out), lambda i, l: (0, 0, 0)),
        ],
        out_specs=pl.BlockSpec((1, n, f_out), lambda i, l: (i, 0, 0)),
        out_shape=jax.ShapeDtypeStruct((b, n, f_out), jnp.float32),
        scratch_shapes=[
            pltpu.VMEM((heads, n, 2 * f_out), jnp.bfloat16),
            pltpu.VMEM((heads, n, 1), jnp.bfloat16),
            pltpu.VMEM((heads, 1, n), jnp.bfloat16),
            pltpu.VMEM((n, n), jnp.bfloat16),
            pltpu.VMEM((n, f_out), jnp.float32),
            pltpu.VMEM((_HEADS, n, n), jnp.bfloat16),
        ],
        compiler_params=pltpu.CompilerParams(
            dimension_semantics=("arbitrary", "arbitrary")),
    )(input_feature, adj, W1, av1, w2, av2)
